# baseline (speedup 1.0000x reference)
import functools

import jax
import jax.numpy as jnp
from jax import lax
from jax.experimental import pallas as pl
from jax.experimental.pallas import tpu as pltpu
from jax.experimental.pallas import tpu_sc as plsc

_IDX_CHUNK = 128


def _make_gather(batch, vocab, dim):
    info = plsc.get_sparse_core_info()
    num_workers = info.num_cores * info.num_subcores
    b_per_w = batch // num_workers
    n_chunks = b_per_w // _IDX_CHUNK
    mesh = plsc.VectorSubcoreMesh(core_axis_name="c", subcore_axis_name="s")

    @functools.partial(
        pl.kernel,
        mesh=mesh,
        out_type=jax.ShapeDtypeStruct((batch, dim), jnp.float32),
        scratch_types=[
            pltpu.VMEM((n_chunks, _IDX_CHUNK), jnp.int32),
            pltpu.VMEM((b_per_w, dim), jnp.float32),
            pltpu.SemaphoreType.DMA,
        ],
        compiler_params=pltpu.CompilerParams(use_tc_tiling_on_sc=False),
    )
    def gather_kernel(table_hbm, idx_hbm, out_hbm, idx_v, rows_v, sem):
        wid = lax.axis_index("s") * info.num_cores + lax.axis_index("c")
        base = wid * b_per_w
        pltpu.sync_copy(idx_hbm.at[pl.ds(wid * n_chunks, n_chunks)], idx_v)
        copies = []
        for j in range(n_chunks):
            copies.append(
                pltpu.async_copy(
                    table_hbm.at[idx_v.at[j]],
                    rows_v.at[pl.ds(j * _IDX_CHUNK, _IDX_CHUNK)],
                    sem,
                )
            )
        for c in copies:
            c.wait()
        pltpu.sync_copy(rows_v, out_hbm.at[pl.ds(base, b_per_w)])

    return gather_kernel


def kernel(variable_hash, embedding_table):
    batch = variable_hash.shape[0]
    vocab, dim = embedding_table.shape
    idx2d = variable_hash.reshape(batch // _IDX_CHUNK, _IDX_CHUNK)
    gather = _make_gather(batch, vocab, dim)
    return gather(embedding_table, idx2d)


# speedup vs baseline: 1.0314x; 1.0314x over previous
import functools

import jax
import jax.numpy as jnp
from jax import lax
from jax.experimental import pallas as pl
from jax.experimental.pallas import tpu as pltpu
from jax.experimental.pallas import tpu_sc as plsc

_LANES = 16


def _make_gather(batch, dim):
    info = plsc.get_sparse_core_info()
    num_workers = info.num_cores * info.num_subcores
    b_per_w = batch // num_workers
    n_bursts = b_per_w // _LANES
    mesh = plsc.VectorSubcoreMesh(core_axis_name="c", subcore_axis_name="s")

    @functools.partial(
        pl.kernel,
        mesh=mesh,
        out_type=jax.ShapeDtypeStruct((batch, dim), jnp.float32),
        scratch_types=[
            pltpu.VMEM((b_per_w,), jnp.int32),
            pltpu.SemaphoreType.DMA,
        ],
        compiler_params=pltpu.CompilerParams(needs_layout_passes=False),
    )
    def gather_kernel(table_hbm, idx_hbm, out_hbm, idx_v, sem):
        wid = lax.axis_index("s") * info.num_cores + lax.axis_index("c")
        base = wid * b_per_w
        pltpu.sync_copy(idx_hbm.at[pl.ds(base, b_per_w)], idx_v)

        lane_ids = lax.iota(jnp.int32, _LANES)
        neg = jnp.full((_LANES,), jnp.iinfo(jnp.int32).min, jnp.int32)

        def burst(k):
            v = idx_v[pl.ds(k * _LANES, _LANES)]
            for l in range(_LANES):
                row = lax.reduce_max(
                    jnp.where(lane_ids == l, v, neg), axes=(0,)
                )
                pltpu.async_copy(
                    table_hbm.at[pl.ds(row, 1)],
                    out_hbm.at[pl.ds(base + k * _LANES + l, 1)],
                    sem,
                )

        pl.loop(0, n_bursts)(burst)

        def drain(_):
            for _ in range(_LANES):
                pltpu.make_async_copy(
                    table_hbm.at[pl.ds(0, 1)],
                    out_hbm.at[pl.ds(base, 1)],
                    sem,
                ).wait()

        pl.loop(0, n_bursts)(drain)

    return gather_kernel


def kernel(variable_hash, embedding_table):
    batch = variable_hash.shape[0]
    dim = embedding_table.shape[1]
    gather = _make_gather(batch, dim)
    return gather(embedding_table, variable_hash)


# speedup vs baseline: 1.7276x; 1.6750x over previous
import functools

import jax
import jax.numpy as jnp
from jax import lax
from jax.experimental import pallas as pl
from jax.experimental.pallas import tpu as pltpu
from jax.experimental.pallas import tpu_sc as plsc

_LANES = 16
_NSEM = 4


def _make_gather(batch, dim):
    info = plsc.get_sparse_core_info()
    num_workers = info.num_cores * info.num_subcores
    b_per_w = batch // num_workers
    n_bursts = b_per_w // _LANES
    mesh = plsc.VectorSubcoreMesh(core_axis_name="c", subcore_axis_name="s")

    @functools.partial(
        pl.kernel,
        mesh=mesh,
        out_type=jax.ShapeDtypeStruct((batch, dim), jnp.float32),
        scratch_types=[
            pltpu.VMEM((b_per_w,), jnp.int32),
            pltpu.VMEM((b_per_w, dim), jnp.float32),
        ]
        + [pltpu.SemaphoreType.DMA] * _NSEM,
        compiler_params=pltpu.CompilerParams(needs_layout_passes=False),
    )
    def gather_kernel(table_hbm, idx_hbm, out_hbm, idx_v, rows_v, *sems):
        wid = lax.axis_index("s") * info.num_cores + lax.axis_index("c")
        base = wid * b_per_w
        pltpu.sync_copy(idx_hbm.at[pl.ds(base, b_per_w)], idx_v)

        lane_ids = lax.iota(jnp.int32, _LANES)
        neg = jnp.full((_LANES,), jnp.iinfo(jnp.int32).min, jnp.int32)

        def burst(k):
            v = idx_v[pl.ds(k * _LANES, _LANES)]
            for l in range(_LANES):
                row = lax.reduce_max(
                    jnp.where(lane_ids == l, v, neg), axes=(0,)
                )
                pltpu.async_copy(
                    table_hbm.at[pl.ds(row, 1)],
                    rows_v.at[pl.ds(k * _LANES + l, 1)],
                    sems[l % _NSEM],
                )

        pl.loop(0, n_bursts)(burst)

        def drain(_):
            for l in range(_LANES):
                pltpu.make_async_copy(
                    table_hbm.at[pl.ds(0, 1)],
                    rows_v.at[pl.ds(0, 1)],
                    sems[l % _NSEM],
                ).wait()

        pl.loop(0, n_bursts)(drain)
        pltpu.sync_copy(rows_v, out_hbm.at[pl.ds(base, b_per_w)])

    return gather_kernel


def kernel(variable_hash, embedding_table):
    batch = variable_hash.shape[0]
    dim = embedding_table.shape[1]
    gather = _make_gather(batch, dim)
    return gather(embedding_table, variable_hash)
